# Initial kernel scaffold; baseline (speedup 1.0000x reference)
import functools

import jax
import jax.numpy as jnp
from jax import lax
from jax.experimental import pallas as pl
from jax.experimental.pallas import tpu as pltpu
from jax.experimental.pallas import tpu_sc as plsc

N_OER, N_CON, N_CLS = 50000, 10000, 1000
NC, NS = 2, 16
K = 128
PAD = {"OER": 53248, "Concept": 12288, "Class": 4096}
F32 = jnp.float32

_LONG = {"ep": "before_ep", "cov": "covers", "bel": "belongs",
         "rcov": "rev_covers", "rbel": "rev_belongs"}


def _rpad(x, n):
    return jnp.pad(x, ((0, n - x.shape[0]),) + ((0, 0),) * (x.ndim - 1))


def _mm_quarters(x, w, bn=2048):
    n = x.shape[0]
    ncb = w.shape[1] // 32

    def body(x_ref, w_ref, o_ref):
        o_ref[0] = jnp.dot(x_ref[...], w_ref[...],
                           preferred_element_type=F32)

    return pl.pallas_call(
        body,
        grid=(n // bn, ncb),
        in_specs=[pl.BlockSpec((bn, 128), lambda i, j: (i, 0)),
                  pl.BlockSpec((128, 32), lambda i, j: (0, j))],
        out_specs=pl.BlockSpec((1, bn, 32), lambda i, j: (j, i, 0)),
        out_shape=jax.ShapeDtypeStruct((ncb, n, 32), F32),
    )(x, w)


def _mm_bias(x, w, b, bn=1024):
    n, k = x.shape

    def body(x_ref, w_ref, b_ref, o_ref):
        o_ref[...] = jnp.dot(x_ref[...], w_ref[...],
                             preferred_element_type=F32) + b_ref[0]

    return pl.pallas_call(
        body,
        grid=(n // bn,),
        in_specs=[pl.BlockSpec((bn, k), lambda i: (i, 0)),
                  pl.BlockSpec((k, 128), lambda i: (0, 0)),
                  pl.BlockSpec((1, 128), lambda i: (0, 0))],
        out_specs=pl.BlockSpec((bn, 128), lambda i: (i, 0)),
        out_shape=jax.ShapeDtypeStruct((n, 128), F32),
    )(x, w, b.reshape(1, 128))


def _edge_w_kernel(n_dst_pad, E, E_pad):
    stripe_e = E_pad // (NC * NS)
    stripe_n = n_dst_pad // NS
    mesh = plsc.VectorSubcoreMesh(core_axis_name="c", subcore_axis_name="s")

    @functools.partial(
        pl.kernel,
        out_type=(jax.ShapeDtypeStruct((2, E_pad), F32),
                  jax.ShapeDtypeStruct((NC * 2 * n_dst_pad,), F32)),
        mesh=mesh,
        scratch_types=[
            pltpu.VMEM((K,), jnp.int32), pltpu.VMEM((K,), jnp.int32),
            pltpu.VMEM((K,), F32), pltpu.VMEM((K,), F32),
            pltpu.VMEM((K,), F32), pltpu.VMEM((K,), F32),
            pltpu.VMEM((K,), F32), pltpu.VMEM((K,), F32),
            pltpu.VMEM_SHARED((n_dst_pad,), F32),
            pltpu.VMEM_SHARED((n_dst_pad,), F32),
            pltpu.VMEM((stripe_n,), F32),
            pltpu.SemaphoreType.DMA,
        ],
    )
    def kern(as0, as1, ad0, ad1, src, dst, zeros1d, w_out, s_out,
             srcv, dstv, a0v, a1v, b0v, b1v, w0v, w1v,
             s0_sh, s1_sh, bounce, sem):
        c = lax.axis_index("c")
        s = lax.axis_index("s")
        wid = c * NS + s
        pltpu.sync_copy(zeros1d.at[pl.ds(0, stripe_n)],
                        s0_sh.at[pl.ds(s * stripe_n, stripe_n)])
        pltpu.sync_copy(zeros1d.at[pl.ds(0, stripe_n)],
                        s1_sh.at[pl.ds(s * stripe_n, stripe_n)])
        plsc.subcore_barrier()

        @pl.loop(0, stripe_e // K)
        def chunk(t):
            base = wid * stripe_e + t * K
            pltpu.sync_copy(src.at[pl.ds(base, K)], srcv)
            pltpu.sync_copy(dst.at[pl.ds(base, K)], dstv)
            pltpu.async_copy(as0.at[srcv], a0v, sem).wait()
            pltpu.async_copy(as1.at[srcv], a1v, sem).wait()
            pltpu.async_copy(ad0.at[dstv], b0v, sem).wait()
            pltpu.async_copy(ad1.at[dstv], b1v, sem).wait()
            for j in range(K // 16):
                sl = pl.ds(j * 16, 16)
                pos = base + j * 16 + lax.iota(jnp.int32, 16)
                valid = pos < E
                x0 = a0v[sl] + b0v[sl]
                x1 = a1v[sl] + b1v[sl]
                w0 = jnp.exp(jnp.maximum(x0, 0.2 * x0))
                w1 = jnp.exp(jnp.maximum(x1, 0.2 * x1))
                w0v[sl] = jnp.where(valid, w0, 0.0)
                w1v[sl] = jnp.where(valid, w1, 0.0)
            pltpu.sync_copy(w0v, w_out.at[0, pl.ds(base, K)])
            pltpu.sync_copy(w1v, w_out.at[1, pl.ds(base, K)])
            pltpu.sync_copy(w0v, s0_sh.at[dstv], add=True)
            pltpu.sync_copy(w1v, s1_sh.at[dstv], add=True)

        plsc.subcore_barrier()
        r0 = s * stripe_n
        pltpu.sync_copy(s0_sh.at[pl.ds(r0, stripe_n)], bounce)
        pltpu.sync_copy(bounce,
                        s_out.at[pl.ds((c * 2 + 0) * n_dst_pad + r0, stripe_n)])
        pltpu.sync_copy(s1_sh.at[pl.ds(r0, stripe_n)], bounce)
        pltpu.sync_copy(bounce,
                        s_out.at[pl.ds((c * 2 + 1) * n_dst_pad + r0, stripe_n)])

    return kern


def _edge_acc_kernel(n_src_pad, n_dst_pad, E_pad, qbase):
    stripe_e = E_pad // NS
    stripe_n = n_dst_pad // NS
    mesh = plsc.VectorSubcoreMesh(core_axis_name="c", subcore_axis_name="s")

    @functools.partial(
        pl.kernel,
        out_type=jax.ShapeDtypeStruct((4, n_dst_pad, 32), F32),
        mesh=mesh,
        scratch_types=[
            pltpu.VMEM((K,), jnp.int32), pltpu.VMEM((K,), jnp.int32),
            pltpu.VMEM((K,), jnp.int32), pltpu.VMEM((K,), F32),
            pltpu.VMEM((K, 32), F32), pltpu.VMEM((K, 32), F32),
            pltpu.VMEM_SHARED((n_dst_pad, 32), F32),
            pltpu.VMEM((256, 32), F32),
            pltpu.SemaphoreType.DMA,
        ],
    )
    def kern(xs_flat, src, dst, w, zeros2d, out,
             srcv, dstv, idxv, wv, rows, msg, acc_sh, bounce, sem):
        c = lax.axis_index("c")
        s = lax.axis_index("s")
        for qq in range(2):
            q = 2 * c + qq
            pltpu.sync_copy(zeros2d.at[pl.ds(s * stripe_n, stripe_n), :],
                            acc_sh.at[pl.ds(s * stripe_n, stripe_n), :])
            plsc.subcore_barrier()
            row_off = (qbase + q) * n_src_pad

            @pl.loop(0, stripe_e // K)
            def chunk(t):
                base = s * stripe_e + t * K
                pltpu.sync_copy(src.at[pl.ds(base, K)], srcv)
                pltpu.sync_copy(dst.at[pl.ds(base, K)], dstv)
                pltpu.sync_copy(w.at[c, pl.ds(base, K)], wv)
                for j in range(K // 16):
                    sl = pl.ds(j * 16, 16)
                    idxv[sl] = srcv[sl] + row_off
                pltpu.async_copy(xs_flat.at[idxv], rows, sem).wait()

                @pl.loop(0, K)
                def edge(e):
                    wsc = wv[e]
                    msg[e, pl.ds(0, 16)] = rows[e, pl.ds(0, 16)] * wsc
                    msg[e, pl.ds(16, 16)] = rows[e, pl.ds(16, 16)] * wsc

                pltpu.sync_copy(msg, acc_sh.at[dstv], add=True)

            plsc.subcore_barrier()

            @pl.loop(0, stripe_n // 256)
            def dmp(tt):
                r0 = s * stripe_n + tt * 256
                pltpu.sync_copy(acc_sh.at[pl.ds(r0, 256), :], bounce)
                pltpu.sync_copy(bounce, out.at[q, pl.ds(r0, 256), :])

            plsc.subcore_barrier()

    return kern


def _edge_cls_kernel(E_pad):
    stripe_e = E_pad // (NC * NS)
    mesh = plsc.VectorSubcoreMesh(core_axis_name="c", subcore_axis_name="s")

    @functools.partial(
        pl.kernel,
        out_type=jax.ShapeDtypeStruct((E_pad,), F32),
        mesh=mesh,
        scratch_types=[
            pltpu.VMEM((K,), jnp.int32), pltpu.VMEM((K,), jnp.int32),
            pltpu.VMEM((K,), F32), pltpu.VMEM((K,), F32),
            pltpu.VMEM((K,), F32),
            pltpu.SemaphoreType.DMA,
        ],
    )
    def kern(u1, u2, src, dst, out, srcv, dstv, g1, g2, ov, sem):
        c = lax.axis_index("c")
        s = lax.axis_index("s")
        wid = c * NS + s

        @pl.loop(0, stripe_e // K)
        def chunk(t):
            base = wid * stripe_e + t * K
            pltpu.sync_copy(src.at[pl.ds(base, K)], srcv)
            pltpu.sync_copy(dst.at[pl.ds(base, K)], dstv)
            pltpu.async_copy(u1.at[srcv], g1, sem).wait()
            pltpu.async_copy(u2.at[dstv], g2, sem).wait()
            for j in range(K // 16):
                sl = pl.ds(j * 16, 16)
                ov[sl] = g1[sl] + g2[sl]
            pltpu.sync_copy(ov, out.at[pl.ds(base, K)])

    return kern


def _finalize(gats, n_pad, bn=1024):
    navg = 1.0 / len(gats)
    specs, args = [], []
    has_self = []
    for g in gats:
        specs += [pl.BlockSpec((4, bn, 32), lambda i: (0, i, 0)),
                  pl.BlockSpec((NC, 2, bn), lambda i: (0, 0, i)),
                  pl.BlockSpec((1, 128), lambda i: (0, 0))]
        args += [g["acc"], g["sp"], g["bias"].reshape(1, 128)]
        has_self.append("xs" in g)
        if "xs" in g:
            specs += [pl.BlockSpec((4, bn, 32), lambda i: (0, i, 0))]
            args += [g["xs"]]
            for t in ("as0", "as1", "ad0", "ad1"):
                specs += [pl.BlockSpec((1, bn), lambda i: (0, i))]
                args += [g[t]]

    def body(*refs):
        o_ref = refs[-1]
        refs = list(refs[:-1])
        total = None
        for self_l in has_self:
            acc_ref, sp_ref, b_ref = refs[:3]
            del refs[:3]
            num = jnp.concatenate([acc_ref[qi] for qi in range(4)], axis=-1)
            s0 = sp_ref[0, 0] + sp_ref[1, 0]
            s1 = sp_ref[0, 1] + sp_ref[1, 1]
            if self_l:
                xs_ref, as0, as1, ad0, ad1 = refs[:5]
                del refs[:5]
                xsq = jnp.concatenate([xs_ref[qi] for qi in range(4)],
                                      axis=-1)
                x0 = as0[0] + ad0[0]
                x1 = as1[0] + ad1[0]
                w0 = jnp.exp(jnp.maximum(x0, 0.2 * x0))
                w1 = jnp.exp(jnp.maximum(x1, 0.2 * x1))
                wcat = jnp.concatenate(
                    [jnp.broadcast_to(w0[:, None], (bn, 64)),
                     jnp.broadcast_to(w1[:, None], (bn, 64))], axis=-1)
                num = num + wcat * xsq
                s0 = s0 + w0
                s1 = s1 + w1
            den = jnp.concatenate(
                [jnp.broadcast_to((s0 + 1e-16)[:, None], (bn, 64)),
                 jnp.broadcast_to((s1 + 1e-16)[:, None], (bn, 64))], axis=-1)
            part = num / den + b_ref[0]
            total = part if total is None else total + part
        o_ref[...] = total * navg

    return pl.pallas_call(
        body,
        grid=(n_pad // bn,),
        in_specs=specs,
        out_specs=pl.BlockSpec((bn, 128), lambda i: (i, 0)),
        out_shape=jax.ShapeDtypeStruct((n_pad, 128), F32),
    )(*args)


def _acol(p, which, h):
    W = p["Wsrc" if which == "s" else "Wdst"]
    att = p["att_src" if which == "s" else "att_dst"][h]
    col = W[:, h * 64:(h + 1) * 64] @ att
    return jnp.pad(col, (0, 128 - col.shape[0]))


def _wpad(W):
    return jnp.pad(W, ((0, 128 - W.shape[0]), (0, 0)))


def _build_wcat(lp):
    z = jnp.zeros((128,), F32)

    def cols(lst):
        return jnp.stack(lst + [z] * (32 - len(lst)), axis=1)

    w_oer = jnp.concatenate([
        _wpad(lp["before_ep"]["Wsrc"]), _wpad(lp["covers"]["Wsrc"]),
        cols([_acol(lp["before_ep"], "s", 0), _acol(lp["before_ep"], "s", 1),
              _acol(lp["before_ep"], "d", 0), _acol(lp["before_ep"], "d", 1),
              _acol(lp["covers"], "s", 0), _acol(lp["covers"], "s", 1),
              _acol(lp["rev_covers"], "d", 0), _acol(lp["rev_covers"], "d", 1)]),
    ], axis=1)
    w_con = jnp.concatenate([
        _wpad(lp["belongs"]["Wsrc"]), _wpad(lp["rev_covers"]["Wsrc"]),
        cols([_acol(lp["covers"], "d", 0), _acol(lp["covers"], "d", 1),
              _acol(lp["belongs"], "s", 0), _acol(lp["belongs"], "s", 1),
              _acol(lp["rev_covers"], "s", 0), _acol(lp["rev_covers"], "s", 1),
              _acol(lp["rev_belongs"], "d", 0), _acol(lp["rev_belongs"], "d", 1)]),
    ], axis=1)
    w_cls = jnp.concatenate([
        _wpad(lp["rev_belongs"]["Wsrc"]),
        cols([_acol(lp["belongs"], "d", 0), _acol(lp["belongs"], "d", 1),
              _acol(lp["rev_belongs"], "s", 0), _acol(lp["rev_belongs"], "s", 1)]),
    ], axis=1)
    return w_oer, w_con, w_cls


def kernel(x_oer, x_concept, x_class, params, ei_sr, ei_ep, ei_cov, ei_bel,
           ei_rcov, ei_rbel):
    NPo, NPc, NPk = PAD["OER"], PAD["Concept"], PAD["Class"]
    zeros2d = jnp.zeros((NPo, 32), F32)
    zeros1d = jnp.zeros((NPo * 32,), F32)

    def pad_edges(ei, mult=4096):
        E = ei.shape[1]
        E_pad = -(-E // mult) * mult
        ei = jnp.pad(ei, ((0, 0), (0, E_pad - E)))
        return ei[0], ei[1], E, E_pad

    edges = {
        "ep": pad_edges(ei_ep) + ("OER", "OER", 0),
        "cov": pad_edges(ei_cov) + ("OER", "Concept", 4),
        "bel": pad_edges(ei_bel) + ("Concept", "Class", 0),
        "rcov": pad_edges(ei_rcov) + ("Concept", "OER", 4),
        "rbel": pad_edges(ei_rbel) + ("Class", "Concept", 0),
    }

    x_pad = {"OER": _rpad(x_oer, NPo), "Concept": _rpad(x_concept, NPc),
             "Class": _rpad(x_class, NPk)}
    h = {}
    for nt in ("OER", "Concept", "Class"):
        W0 = jnp.pad(params["lin"][nt]["W"], ((0, 0), (0, 64)))
        b0 = jnp.pad(params["lin"][nt]["b"], (0, 64))
        h[nt] = _mm_bias(x_pad[nt], W0, b0)

    acol_off = {
        "ep": ("OER", 0, "OER", 2), "cov": ("OER", 4, "Concept", 0),
        "bel": ("Concept", 2, "Class", 0), "rcov": ("Concept", 4, "OER", 6),
        "rbel": ("Class", 2, "Concept", 6),
    }

    for lp in params["layers"]:
        w_oer, w_con, w_cls = _build_wcat(lp)
        out_cat = {
            "OER": _mm_quarters(h["OER"], w_oer),
            "Concept": _mm_quarters(h["Concept"], w_con),
            "Class": _mm_quarters(h["Class"], w_cls),
        }
        acolb = {nt: out_cat[nt][out_cat[nt].shape[0] - 1] for nt in out_cat}
        xs_flat = {nt: out_cat[nt].reshape(-1, 32) for nt in out_cat}

        res = {}
        for name, (src, dst, E, E_pad, st, dt, qbase) in edges.items():
            _, aso, _, ado = acol_off[name]
            as0 = acolb[st][:, aso]
            as1 = acolb[st][:, aso + 1]
            ad0 = acolb[dt][:, ado]
            ad1 = acolb[dt][:, ado + 1]
            n_dst_pad = PAD[dt]
            n_src_pad = PAD[st]
            wk = _edge_w_kernel(n_dst_pad, E, E_pad)
            w_e, s_part = wk(as0, as1, ad0, ad1, src, dst, zeros1d)
            ak = _edge_acc_kernel(n_src_pad, n_dst_pad, E_pad, qbase)
            acc = ak(xs_flat[st], src, dst, w_e, zeros2d)
            res[name] = {"acc": acc, "sp": s_part.reshape(NC, 2, n_dst_pad),
                         "bias": lp[_LONG[name]]["bias"]}

        res["ep"]["xs"] = out_cat["OER"][:4]
        res["ep"]["as0"] = acolb["OER"][:, 0].reshape(1, -1)
        res["ep"]["as1"] = acolb["OER"][:, 1].reshape(1, -1)
        res["ep"]["ad0"] = acolb["OER"][:, 2].reshape(1, -1)
        res["ep"]["ad1"] = acolb["OER"][:, 3].reshape(1, -1)

        h = {
            "OER": _finalize([res["ep"], res["rcov"]], NPo),
            "Concept": _finalize([res["cov"], res["rbel"]], NPc),
            "Class": _finalize([res["bel"]], NPk),
        }

    Wc = params["cls"]["W"]
    bc = params["cls"]["b"]
    wmat = jnp.zeros((256, 128), F32)
    wmat = wmat.at[:, 0].set(Wc[:256, 0]).at[:, 1].set(Wc[256:, 0])
    bvec = jnp.zeros((128,), F32).at[0].set(bc[0])
    xcat = jnp.concatenate([x_pad["OER"], h["OER"]], axis=1)
    u = _mm_bias(xcat, wmat, bvec)
    u1 = u[:, 0]
    u2 = u[:, 1]
    src, dst, E, E_pad = pad_edges(ei_sr)
    ck = _edge_cls_kernel(E_pad)
    pred = ck(u1, u2, src, dst)
    return pred[:E]


# speedup vs baseline: 18.3215x; 18.3215x over previous
import functools

import jax
import jax.numpy as jnp
from jax import lax
from jax.experimental import pallas as pl
from jax.experimental.pallas import tpu as pltpu
from jax.experimental.pallas import tpu_sc as plsc

NC, NS = 2, 16
K = 128
PAD = {"OER": 55296, "Concept": 10240, "Class": 4096}
NR = 6
RSZ = 9216
ACC_ROWS = 10240
F32 = jnp.float32

_LONG = {"ep": "before_ep", "cov": "covers", "bel": "belongs",
         "rcov": "rev_covers", "rbel": "rev_belongs"}


def _rpad(x, n):
    return jnp.pad(x, ((0, n - x.shape[0]),) + ((0, 0),) * (x.ndim - 1))


def _mm_bias(x, w, b, bn=1024):
    n, k = x.shape

    def body(x_ref, w_ref, b_ref, o_ref):
        o_ref[...] = jnp.dot(x_ref[...], w_ref[...],
                             preferred_element_type=F32) + b_ref[0]

    return pl.pallas_call(
        body,
        grid=(n // bn,),
        in_specs=[pl.BlockSpec((bn, k), lambda i: (i, 0)),
                  pl.BlockSpec((k, 128), lambda i: (0, 0)),
                  pl.BlockSpec((1, 128), lambda i: (0, 0))],
        out_specs=pl.BlockSpec((bn, 128), lambda i: (i, 0)),
        out_shape=jax.ShapeDtypeStruct((n, 128), F32),
    )(x, w, b.reshape(1, 128))


def _edge_w_kernel(n_dst_pad, E, E_pad):
    stripe_e = E_pad // (NC * NS)
    stripe_n = n_dst_pad // NS
    mesh = plsc.VectorSubcoreMesh(core_axis_name="c", subcore_axis_name="s")

    @functools.partial(
        pl.kernel,
        out_type=(jax.ShapeDtypeStruct((E_pad,), F32),
                  jax.ShapeDtypeStruct((E_pad,), F32),
                  jax.ShapeDtypeStruct((NC * 2 * n_dst_pad,), F32)),
        mesh=mesh,
        scratch_types=[
            pltpu.VMEM((K,), jnp.int32), pltpu.VMEM((K,), jnp.int32),
            pltpu.VMEM((K,), F32), pltpu.VMEM((K,), F32),
            pltpu.VMEM((K,), F32), pltpu.VMEM((K,), F32),
            pltpu.VMEM((K,), F32), pltpu.VMEM((K,), F32),
            pltpu.VMEM_SHARED((n_dst_pad,), F32),
            pltpu.VMEM_SHARED((n_dst_pad,), F32),
            pltpu.VMEM((stripe_n,), F32),
            pltpu.SemaphoreType.DMA,
        ],
    )
    def kern(as0, as1, ad0, ad1, src, dst, zeros1d, w0_out, w1_out, s_out,
             srcv, dstv, a0v, a1v, b0v, b1v, w0v, w1v,
             s0_sh, s1_sh, bounce, sem):
        c = lax.axis_index("c")
        s = lax.axis_index("s")
        wid = c * NS + s
        pltpu.sync_copy(zeros1d.at[pl.ds(0, stripe_n)], bounce)
        pltpu.sync_copy(bounce, s0_sh.at[pl.ds(s * stripe_n, stripe_n)])
        pltpu.sync_copy(bounce, s1_sh.at[pl.ds(s * stripe_n, stripe_n)])
        plsc.subcore_barrier()

        @pl.loop(0, stripe_e // K)
        def chunk(t):
            base = wid * stripe_e + t * K
            pltpu.sync_copy(src.at[pl.ds(base, K)], srcv)
            pltpu.sync_copy(dst.at[pl.ds(base, K)], dstv)
            pltpu.async_copy(as0.at[srcv], a0v, sem).wait()
            pltpu.async_copy(as1.at[srcv], a1v, sem).wait()
            pltpu.async_copy(ad0.at[dstv], b0v, sem).wait()
            pltpu.async_copy(ad1.at[dstv], b1v, sem).wait()
            for j in range(K // 16):
                sl = pl.ds(j * 16, 16)
                pos = base + j * 16 + lax.iota(jnp.int32, 16)
                valid = pos < E
                x0 = a0v[sl] + b0v[sl]
                x1 = a1v[sl] + b1v[sl]
                w0 = jnp.exp(jnp.maximum(x0, 0.2 * x0))
                w1 = jnp.exp(jnp.maximum(x1, 0.2 * x1))
                w0v[sl] = jnp.where(valid, w0, 0.0)
                w1v[sl] = jnp.where(valid, w1, 0.0)
            pltpu.sync_copy(w0v, w0_out.at[pl.ds(base, K)])
            pltpu.sync_copy(w1v, w1_out.at[pl.ds(base, K)])
            pltpu.sync_copy(w0v, s0_sh.at[dstv], add=True)
            pltpu.sync_copy(w1v, s1_sh.at[dstv], add=True)

        plsc.subcore_barrier()
        r0 = s * stripe_n
        pltpu.sync_copy(s0_sh.at[pl.ds(r0, stripe_n)], bounce)
        pltpu.sync_copy(bounce,
                        s_out.at[pl.ds((c * 2 + 0) * n_dst_pad + r0, stripe_n)])
        pltpu.sync_copy(s1_sh.at[pl.ds(r0, stripe_n)], bounce)
        pltpu.sync_copy(bounce,
                        s_out.at[pl.ds((c * 2 + 1) * n_dst_pad + r0, stripe_n)])

    return kern


def _edge_acc_kernel(n_dst_pad, E_pad, ranged):
    mesh = plsc.VectorSubcoreMesh(core_axis_name="c", subcore_axis_name="s")
    if ranged:
        acc_rows = ACC_ROWS
        out_ty = jax.ShapeDtypeStruct((n_dst_pad, 128), F32)
        stripe_e = E_pad // NS
    else:
        acc_rows = n_dst_pad
        out_ty = jax.ShapeDtypeStruct((NC, n_dst_pad, 128), F32)
        stripe_e = E_pad // (NC * NS)
    zstripe = acc_rows // NS

    @functools.partial(
        pl.kernel,
        out_type=out_ty,
        mesh=mesh,
        scratch_types=[
            pltpu.VMEM((K,), jnp.int32), pltpu.VMEM((K,), jnp.int32),
            pltpu.VMEM((K,), jnp.int32),
            pltpu.VMEM((K,), F32), pltpu.VMEM((K,), F32),
            pltpu.VMEM((K, 128), F32), pltpu.VMEM((K, 128), F32),
            pltpu.VMEM_SHARED((acc_rows, 128), F32),
            pltpu.VMEM((64, 128), F32),
            pltpu.SemaphoreType.DMA,
        ],
    )
    def kern(xs, src, dst, w0, w1, out,
             srcv, dstv, ldst, w0v, w1v, rows, msg, acc_sh, zbuf, sem):
        c = lax.axis_index("c")
        s = lax.axis_index("s")

        @pl.loop(0, 64)
        def zfill(r):
            for j in range(8):
                zbuf[r, pl.ds(j * 16, 16)] = jnp.zeros((16,), F32)

        def run_pass(base_row, ebase, out_view):
            @pl.loop(0, zstripe // 64)
            def zrow(zz):
                pltpu.sync_copy(
                    zbuf, acc_sh.at[pl.ds(s * zstripe + zz * 64, 64), :])

            plsc.subcore_barrier()

            @pl.loop(0, stripe_e // K)
            def chunk(t):
                base = ebase + s * stripe_e + t * K
                pltpu.sync_copy(src.at[pl.ds(base, K)], srcv)
                pltpu.sync_copy(dst.at[pl.ds(base, K)], dstv)
                pltpu.sync_copy(w0.at[pl.ds(base, K)], w0v)
                pltpu.sync_copy(w1.at[pl.ds(base, K)], w1v)
                if ranged:
                    for j in range(K // 16):
                        sl = pl.ds(j * 16, 16)
                        lv = dstv[sl] - base_row
                        ok = (lv >= 0) & (lv < RSZ)
                        ldst[sl] = jnp.where(ok, lv, RSZ + s)
                pltpu.async_copy(xs.at[srcv], rows, sem).wait()

                @pl.loop(0, K // 16)
                def edge_grp(g):
                    w0vec = w0v[pl.ds(g * 16, 16)]
                    w1vec = w1v[pl.ds(g * 16, 16)]
                    for e16 in range(16):
                        e = g * 16 + e16
                        ws0 = w0vec[e16]
                        ws1 = w1vec[e16]
                        for j in range(4):
                            slc = pl.ds(j * 16, 16)
                            msg[e, slc] = rows[e, slc] * ws0
                        for j in range(4, 8):
                            slc = pl.ds(j * 16, 16)
                            msg[e, slc] = rows[e, slc] * ws1

                pltpu.sync_copy(msg, acc_sh.at[ldst if ranged else dstv],
                                add=True)

            plsc.subcore_barrier()
            nreal = RSZ if ranged else n_dst_pad
            per_tile = nreal // NS

            @pl.loop(0, per_tile // 64)
            def dmp(tt):
                r0 = s * per_tile + tt * 64
                pltpu.sync_copy(acc_sh.at[pl.ds(r0, 64), :], msg.at[pl.ds(0, 64), :])
                pltpu.sync_copy(msg.at[pl.ds(0, 64), :],
                                out_view.at[pl.ds(base_row + r0, 64), :])

            plsc.subcore_barrier()

        if ranged:
            for rr in range(NR // NC):
                run_pass((c * (NR // NC) + rr) * RSZ, 0, out)
        else:
            run_pass(0, c * (NS * stripe_e), out.at[c])

    return kern


def _edge_cls_kernel(E_pad):
    stripe_e = E_pad // (NC * NS)
    mesh = plsc.VectorSubcoreMesh(core_axis_name="c", subcore_axis_name="s")

    @functools.partial(
        pl.kernel,
        out_type=jax.ShapeDtypeStruct((E_pad,), F32),
        mesh=mesh,
        scratch_types=[
            pltpu.VMEM((K,), jnp.int32), pltpu.VMEM((K,), jnp.int32),
            pltpu.VMEM((K,), F32), pltpu.VMEM((K,), F32),
            pltpu.VMEM((K,), F32),
            pltpu.SemaphoreType.DMA,
        ],
    )
    def kern(u1, u2, src, dst, out, srcv, dstv, g1, g2, ov, sem):
        c = lax.axis_index("c")
        s = lax.axis_index("s")
        wid = c * NS + s

        @pl.loop(0, stripe_e // K)
        def chunk(t):
            base = wid * stripe_e + t * K
            pltpu.sync_copy(src.at[pl.ds(base, K)], srcv)
            pltpu.sync_copy(dst.at[pl.ds(base, K)], dstv)
            pltpu.async_copy(u1.at[srcv], g1, sem).wait()
            pltpu.async_copy(u2.at[dstv], g2, sem).wait()
            for j in range(K // 16):
                sl = pl.ds(j * 16, 16)
                ov[sl] = g1[sl] + g2[sl]
            pltpu.sync_copy(ov, out.at[pl.ds(base, K)])

    return kern


def _finalize(gats, n_pad, bn=1024):
    navg = 1.0 / len(gats)
    specs, args, has_self, split_acc = [], [], [], []
    for g in gats:
        if g["acc"].ndim == 3:
            specs.append(pl.BlockSpec((NC, bn, 128), lambda i: (0, i, 0)))
            split_acc.append(True)
        else:
            specs.append(pl.BlockSpec((bn, 128), lambda i: (i, 0)))
            split_acc.append(False)
        specs += [pl.BlockSpec((NC, 2, bn), lambda i: (0, 0, i)),
                  pl.BlockSpec((1, 128), lambda i: (0, 0))]
        args += [g["acc"], g["sp"], g["bias"].reshape(1, 128)]
        has_self.append("xs" in g)
        if "xs" in g:
            specs.append(pl.BlockSpec((bn, 128), lambda i: (i, 0)))
            args.append(g["xs"])
            for t in ("as0", "as1", "ad0", "ad1"):
                specs.append(pl.BlockSpec((1, bn), lambda i: (0, i)))
                args.append(g[t])

    def body(*refs):
        o_ref = refs[-1]
        refs = list(refs[:-1])
        total = None
        for self_l, sp_acc in zip(has_self, split_acc):
            acc_ref, sp_ref, b_ref = refs[:3]
            del refs[:3]
            num = acc_ref[0] + acc_ref[1] if sp_acc else acc_ref[...]
            s0 = sp_ref[0, 0] + sp_ref[1, 0]
            s1 = sp_ref[0, 1] + sp_ref[1, 1]
            if self_l:
                xs_ref, as0, as1, ad0, ad1 = refs[:5]
                del refs[:5]
                x0 = as0[0] + ad0[0]
                x1 = as1[0] + ad1[0]
                w0 = jnp.exp(jnp.maximum(x0, 0.2 * x0))
                w1 = jnp.exp(jnp.maximum(x1, 0.2 * x1))
                wcat = jnp.concatenate(
                    [jnp.broadcast_to(w0[:, None], (bn, 64)),
                     jnp.broadcast_to(w1[:, None], (bn, 64))], axis=-1)
                num = num + wcat * xs_ref[...]
                s0 = s0 + w0
                s1 = s1 + w1
            den = jnp.concatenate(
                [jnp.broadcast_to((s0 + 1e-16)[:, None], (bn, 64)),
                 jnp.broadcast_to((s1 + 1e-16)[:, None], (bn, 64))], axis=-1)
            part = num / den + b_ref[0]
            total = part if total is None else total + part
        o_ref[...] = total * navg

    return pl.pallas_call(
        body,
        grid=(n_pad // bn,),
        in_specs=specs,
        out_specs=pl.BlockSpec((bn, 128), lambda i: (i, 0)),
        out_shape=jax.ShapeDtypeStruct((n_pad, 128), F32),
    )(*args)


def _acol(p, which, h):
    W = p["Wsrc" if which == "s" else "Wdst"]
    att = p["att_src" if which == "s" else "att_dst"][h]
    col = W[:, h * 64:(h + 1) * 64] @ att
    return jnp.pad(col, (0, 128 - col.shape[0]))


def _wpad(W):
    return jnp.pad(W, ((0, 128 - W.shape[0]), (0, 0)))


def kernel(x_oer, x_concept, x_class, params, ei_sr, ei_ep, ei_cov, ei_bel,
           ei_rcov, ei_rbel):
    NPo, NPc, NPk = PAD["OER"], PAD["Concept"], PAD["Class"]
    zeros1d = jnp.zeros((NPo // NS,), F32)
    zero_b = jnp.zeros((128,), F32)

    def pad_edges(ei, mult=4096):
        E = ei.shape[1]
        E_pad = -(-E // mult) * mult
        ei = jnp.pad(ei, ((0, 0), (0, E_pad - E)))
        return ei[0], ei[1], E, E_pad

    edges = {
        "ep": pad_edges(ei_ep) + ("OER", "OER"),
        "cov": pad_edges(ei_cov) + ("OER", "Concept"),
        "bel": pad_edges(ei_bel) + ("Concept", "Class"),
        "rcov": pad_edges(ei_rcov) + ("Concept", "OER"),
        "rbel": pad_edges(ei_rbel) + ("Class", "Concept"),
    }

    x_pad = {"OER": _rpad(x_oer, NPo), "Concept": _rpad(x_concept, NPc),
             "Class": _rpad(x_class, NPk)}
    h = {}
    for nt in ("OER", "Concept", "Class"):
        W0 = jnp.pad(params["lin"][nt]["W"], ((0, 0), (0, 64)))
        b0 = jnp.pad(params["lin"][nt]["b"], (0, 64))
        h[nt] = _mm_bias(x_pad[nt], W0, b0)

    acol_cols = {
        "OER": [("ep", "s"), ("ep", "d"), ("cov", "s"), ("rcov", "d")],
        "Concept": [("cov", "d"), ("bel", "s"), ("rcov", "s"), ("rbel", "d")],
        "Class": [("bel", "d"), ("rbel", "s")],
    }

    for lp in params["layers"]:
        xs = {name: _mm_bias(h[st], _wpad(lp[_LONG[name]]["Wsrc"]), zero_b)
              for name, (_, _, _, _, st, _) in edges.items()}
        atab = {}
        for nt, colspec in acol_cols.items():
            cols = []
            for gname, which in colspec:
                cols += [_acol(lp[_LONG[gname]], which, 0),
                         _acol(lp[_LONG[gname]], which, 1)]
            wa = jnp.stack(cols + [jnp.zeros((128,), F32)] *
                           (128 - len(cols)), axis=1)
            am = _mm_bias(h[nt], wa, zero_b)
            for i2, (gname, which) in enumerate(colspec):
                atab[(gname, which, 0)] = am[:, 2 * i2]
                atab[(gname, which, 1)] = am[:, 2 * i2 + 1]

        res = {}
        for name, (src, dst, E, E_pad, st, dt) in edges.items():
            n_dst_pad = PAD[dt]
            wk = _edge_w_kernel(n_dst_pad, E, E_pad)
            w0_e, w1_e, s_part = wk(atab[(name, "s", 0)], atab[(name, "s", 1)],
                                    atab[(name, "d", 0)], atab[(name, "d", 1)],
                                    src, dst, zeros1d)
            ak = _edge_acc_kernel(n_dst_pad, E_pad, ranged=(dt == "OER"))
            acc = ak(xs[name], src, dst, w0_e, w1_e)
            res[name] = {"acc": acc, "sp": s_part.reshape(NC, 2, n_dst_pad),
                         "bias": lp[_LONG[name]]["bias"]}

        res["ep"]["xs"] = xs["ep"]
        res["ep"]["as0"] = atab[("ep", "s", 0)].reshape(1, -1)
        res["ep"]["as1"] = atab[("ep", "s", 1)].reshape(1, -1)
        res["ep"]["ad0"] = atab[("ep", "d", 0)].reshape(1, -1)
        res["ep"]["ad1"] = atab[("ep", "d", 1)].reshape(1, -1)

        h = {
            "OER": _finalize([res["ep"], res["rcov"]], NPo),
            "Concept": _finalize([res["cov"], res["rbel"]], NPc),
            "Class": _finalize([res["bel"]], NPk),
        }

    Wc = params["cls"]["W"]
    bc = params["cls"]["b"]
    wmat = jnp.zeros((256, 128), F32)
    wmat = wmat.at[:, 0].set(Wc[:256, 0]).at[:, 1].set(Wc[256:, 0])
    bvec = jnp.zeros((128,), F32).at[0].set(bc[0])
    xcat = jnp.concatenate([x_pad["OER"], h["OER"]], axis=1)
    u = _mm_bias(xcat, wmat, bvec)
    u1 = u[:, 0]
    u2 = u[:, 1]
    src, dst, E, E_pad = pad_edges(ei_sr)
    ck = _edge_cls_kernel(E_pad)
    pred = ck(u1, u2, src, dst)
    return pred[:E]
